# Initial kernel scaffold; baseline (speedup 1.0000x reference)
import jax
import jax.numpy as jnp
from jax.experimental import pallas as pl


def kernel(traj, mat, traj_len, emb_t, emb_l, emb_u, emb_su, emb_sl, emb_tu, emb_tl):
    raise NotImplementedError("write your pallas kernel here")


# speedup vs baseline: 13.9282x; 13.9282x over previous
import functools

import jax
import jax.numpy as jnp
from jax import lax
from jax.experimental import pallas as pl
from jax.experimental.pallas import tpu as pltpu
from jax.experimental.pallas import tpu_sc as plsc

HOURS = 24 * 7
SU, SL, TU, TL = 1000.0, 0.0, 500.0, 0.0
B, L, D = 1024, 20, 64
N = B * L
NW = 32
ROWS_W = N // NW
CHUNK = 128
NCH = ROWS_W // CHUNK


def _joint_sc(emb_t_hbm, emb_l_hbm, emb_u_hbm, idx_t_hbm, idx_l_hbm,
              idx_u_hbm, out_hbm, idx_t_v, idx_l_v, idx_u_v,
              rows_t, rows_l, rows_u, sem):
    wid = lax.axis_index("s") * 2 + lax.axis_index("c")
    base = wid * ROWS_W

    pltpu.sync_copy(idx_t_hbm.at[wid], idx_t_v)
    pltpu.sync_copy(idx_l_hbm.at[wid], idx_l_v)
    pltpu.sync_copy(idx_u_hbm.at[wid], idx_u_v)

    copies = []
    for tbl, idx_v, rows in ((emb_t_hbm, idx_t_v, rows_t),
                             (emb_l_hbm, idx_l_v, rows_l),
                             (emb_u_hbm, idx_u_v, rows_u)):
        for j in range(NCH):
            cp = pltpu.make_async_copy(
                tbl.at[idx_v.at[j]], rows.at[pl.ds(j * CHUNK, CHUNK)], sem)
            cp.start()
            copies.append(cp)
    for cp in copies:
        cp.wait()

    def body(i, _):
        for c in range(D // 16):
            s = pl.ds(c * 16, 16)
            rows_l[i, s] = rows_l[i, s] + rows_u[i, s] + rows_t[i, s]
        return 0

    lax.fori_loop(0, ROWS_W, body, 0)
    pltpu.sync_copy(rows_l, out_hbm.at[pl.ds(base, ROWS_W)])


@functools.partial(jax.jit, static_argnums=())
def _joint(emb_t, emb_l, emb_u, idx_t, idx_l, idx_u):
    mesh = plsc.VectorSubcoreMesh(core_axis_name="c", subcore_axis_name="s")
    return pl.kernel(
        _joint_sc,
        out_type=jax.ShapeDtypeStruct((N, D), jnp.float32),
        mesh=mesh,
        compiler_params=pltpu.CompilerParams(use_tc_tiling_on_sc=False),
        scratch_types=[
            pltpu.VMEM((NCH, CHUNK), jnp.int32),
            pltpu.VMEM((NCH, CHUNK), jnp.int32),
            pltpu.VMEM((NCH, CHUNK), jnp.int32),
            pltpu.VMEM((ROWS_W, D), jnp.float32),
            pltpu.VMEM((ROWS_W, D), jnp.float32),
            pltpu.VMEM((ROWS_W, D), jnp.float32),
            pltpu.SemaphoreType.DMA,
        ],
    )(emb_t, emb_l, emb_u, idx_t, idx_l, idx_u)


BB = 16


def _delta_body(len_ref, ds_ref, dt_ref, su_ref, sl_ref, tu_ref, tl_ref,
                out_ref):
    lens = len_ref[...]
    pos = lax.broadcasted_iota(jnp.int32, (1, L * L), 1)
    m = ((pos // L) < lens) & ((pos % L) < lens)
    sl = sl_ref[...]
    su = su_ref[...]
    tl = tl_ref[...]
    tu = tu_ref[...]
    a0 = sl[0] + tl[0]
    a1 = sl[1] + tl[1]
    b0 = (su[0] - sl[0]) * (1.0 / (SU - SL))
    b1 = (su[1] - sl[1]) * (1.0 / (SU - SL))
    c0 = (tu[0] - tl[0]) * (1.0 / (TU - TL))
    c1 = (tu[1] - tl[1]) * (1.0 / (TU - TL))
    mf = m.astype(jnp.float32)[..., None]
    ds = ds_ref[...][..., None]
    dt = dt_ref[...][..., None]
    r0 = a0 + ds * b0 + dt * c0
    r1 = a1 + ds * b1 + dt * c1
    out_ref[...] = r0 + mf * (r1 - r0)


def _delta(traj_len2d, ds, dt, emb_su, emb_sl, emb_tu, emb_tl):
    grid = (B // BB,)
    small = pl.BlockSpec((2, D), lambda i: (0, 0))
    return pl.pallas_call(
        _delta_body,
        grid=grid,
        in_specs=[
            pl.BlockSpec((BB, 1), lambda i: (i, 0)),
            pl.BlockSpec((BB, L * L), lambda i: (i, 0)),
            pl.BlockSpec((BB, L * L), lambda i: (i, 0)),
            small, small, small, small,
        ],
        out_specs=pl.BlockSpec((BB, L * L, D), lambda i: (i, 0, 0)),
        out_shape=jax.ShapeDtypeStruct((B, L * L, D), jnp.float32),
    )(traj_len2d, ds, dt, emb_su, emb_sl, emb_tu, emb_tl)


def kernel(traj, mat, traj_len, emb_t, emb_l, emb_u, emb_su, emb_sl,
           emb_tu, emb_tl):
    idx_t = ((traj[:, :, 2] - 1) % HOURS + 1).reshape(NW, NCH, CHUNK)
    idx_l = traj[:, :, 1].reshape(NW, NCH, CHUNK)
    idx_u = traj[:, :, 0].reshape(NW, NCH, CHUNK)
    joint = _joint(emb_t, emb_l, emb_u, idx_t, idx_l, idx_u).reshape(B, L, D)

    ds = mat[:, :, :, 0].reshape(B, L * L)
    dt = mat[:, :, :, 1].reshape(B, L * L)
    delta = _delta(traj_len.reshape(B, 1), ds, dt, emb_su, emb_sl,
                   emb_tu, emb_tl).reshape(B, L, L, D)
    return (joint, delta)


# speedup vs baseline: 22.7322x; 1.6321x over previous
import functools

import jax
import jax.numpy as jnp
from jax import lax
from jax.experimental import pallas as pl
from jax.experimental.pallas import tpu as pltpu
from jax.experimental.pallas import tpu_sc as plsc

HOURS = 24 * 7
SU, SL, TU, TL = 1000.0, 0.0, 500.0, 0.0
B, L, D = 1024, 20, 64
V = 100000
N = B * L
NW = 32
ROWS_W = N // NW
CHUNK = 128
NCH = ROWS_W // CHUNK


def _joint_sc(emb_lu_hbm, emb_t_hbm, idx_t_hbm, idx_l_hbm, idx_u_hbm,
              out_hbm, idx_t_v, idx_l_v, idx_u_v, gl, gu, gt, acc, sem):
    wid = lax.axis_index("s") * 2 + lax.axis_index("c")

    pltpu.sync_copy(idx_t_hbm.at[wid], idx_t_v)
    pltpu.sync_copy(idx_l_hbm.at[wid], idx_l_v)
    pltpu.sync_copy(idx_u_hbm.at[wid], idx_u_v)

    def fire(c):
        sl_ = c & 1
        pltpu.make_async_copy(emb_lu_hbm.at[idx_l_v.at[c]], gl.at[sl_], sem
                              ).start()
        pltpu.make_async_copy(emb_lu_hbm.at[idx_u_v.at[c]], gu.at[sl_], sem
                              ).start()
        pltpu.make_async_copy(emb_t_hbm.at[idx_t_v.at[c]], gt.at[sl_], sem
                              ).start()

    def drain(c):
        sl_ = c & 1
        pltpu.make_async_copy(emb_lu_hbm.at[idx_l_v.at[c]], gl.at[sl_], sem
                              ).wait()
        pltpu.make_async_copy(emb_lu_hbm.at[idx_u_v.at[c]], gu.at[sl_], sem
                              ).wait()
        pltpu.make_async_copy(emb_t_hbm.at[idx_t_v.at[c]], gt.at[sl_], sem
                              ).wait()

    fire(0)
    for c in range(NCH):
        if c + 1 < NCH:
            fire(c + 1)
        drain(c)
        sl_ = c & 1

        def body(q, _):
            r0 = 2 * q
            r1 = 2 * q + 1
            ac = c * (CHUNK // 2) + q
            for cc in range(D // 16):
                s = pl.ds(cc * 16, 16)
                su_ = pl.ds(D + cc * 16, 16)
                acc[ac, s] = gl[sl_, r0, s] + gu[sl_, r0, su_] + gt[sl_, r0, s]
                acc[ac, pl.ds(D + cc * 16, 16)] = (
                    gl[sl_, r1, s] + gu[sl_, r1, su_] + gt[sl_, r1, s])
            return 0

        lax.fori_loop(0, CHUNK // 2, body, 0)

    pltpu.sync_copy(acc, out_hbm.at[pl.ds(wid * (ROWS_W // 2), ROWS_W // 2)])


def _joint(emb_lu, emb_t, idx_t, idx_l, idx_u):
    mesh = plsc.VectorSubcoreMesh(core_axis_name="c", subcore_axis_name="s")
    return pl.kernel(
        _joint_sc,
        out_type=jax.ShapeDtypeStruct((N // 2, 2 * D), jnp.float32),
        mesh=mesh,
        compiler_params=pltpu.CompilerParams(use_tc_tiling_on_sc=False),
        scratch_types=[
            pltpu.VMEM((NCH, CHUNK), jnp.int32),
            pltpu.VMEM((NCH, CHUNK), jnp.int32),
            pltpu.VMEM((NCH, CHUNK), jnp.int32),
            pltpu.VMEM((2, CHUNK, 2 * D), jnp.float32),
            pltpu.VMEM((2, CHUNK, 2 * D), jnp.float32),
            pltpu.VMEM((2, CHUNK, D), jnp.float32),
            pltpu.VMEM((ROWS_W // 2, 2 * D), jnp.float32),
            pltpu.SemaphoreType.DMA,
        ],
    )(emb_lu, emb_t, idx_t, idx_l, idx_u)


BB = 32
Q = L * L // 2


def _delta_body(lens_ref, dsE_ref, dtE_ref, dsO_ref, dtO_ref,
                su_ref, sl_ref, tu_ref, tl_ref, out_ref):
    sl = sl_ref[...]
    su = su_ref[...]
    tl = tl_ref[...]
    tu = tu_ref[...]
    a0 = (sl[0] + tl[0])[None, :]
    da = (sl[1] + tl[1])[None, :] - a0
    b0 = ((su[0] - sl[0]) * (1.0 / (SU - SL)))[None, :]
    db = ((su[1] - sl[1]) * (1.0 / (SU - SL)))[None, :] - b0
    c0 = ((tu[0] - tl[0]) * (1.0 / (TU - TL)))[None, :]
    dc = ((tu[1] - tl[1]) * (1.0 / (TU - TL)))[None, :] - c0
    zz = jnp.zeros((2, D), jnp.float32)
    vtop = jnp.concatenate([a0, b0, c0, da, db, dc, zz], axis=0)
    z8 = jnp.zeros((8, D), jnp.float32)
    v2 = jnp.concatenate(
        [jnp.concatenate([vtop, z8], axis=1),
         jnp.concatenate([z8, vtop], axis=1)], axis=0
    ).astype(jnp.bfloat16)

    q = lax.broadcasted_iota(jnp.int32, (1, Q), 1)
    pe = 2 * q
    po = 2 * q + 1
    ie, je = pe // L, pe % L
    io, jo = po // L, po % L
    ones = jnp.ones((1, Q), jnp.float32)
    zrow = jnp.zeros((2, Q), jnp.float32)
    lens = lens_ref[...]
    for b in range(BB):
        lb = lens[b:b + 1, 0:1]
        me = ((ie < lb) & (je < lb)).astype(jnp.float32)
        mo = ((io < lb) & (jo < lb)).astype(jnp.float32)
        dse = dsE_ref[b:b + 1, :]
        dte = dtE_ref[b:b + 1, :]
        dso = dsO_ref[b:b + 1, :]
        dto = dtO_ref[b:b + 1, :]
        ct = jnp.concatenate(
            [ones, dse, dte, me, me * dse, me * dte, zrow,
             ones, dso, dto, mo, mo * dso, mo * dto, zrow], axis=0
        ).astype(jnp.bfloat16)
        out_ref[b] = lax.dot_general(
            ct, v2, (((0,), (0,)), ((), ())),
            preferred_element_type=jnp.float32)


def _delta(lens, dsE, dtE, dsO, dtO, emb_su, emb_sl, emb_tu, emb_tl):
    grid = (B // BB,)
    small = pl.BlockSpec((2, D), lambda i: (0, 0))
    coef = pl.BlockSpec((BB, Q), lambda i: (i, 0))
    return pl.pallas_call(
        _delta_body,
        grid=grid,
        in_specs=[
            pl.BlockSpec((BB, 1), lambda i: (i, 0)),
            coef, coef, coef, coef,
            small, small, small, small,
        ],
        out_specs=pl.BlockSpec((BB, Q, 2 * D), lambda i: (i, 0, 0)),
        out_shape=jax.ShapeDtypeStruct((B, Q, 2 * D), jnp.float32),
    )(lens, dsE, dtE, dsO, dtO, emb_su, emb_sl, emb_tu, emb_tl)


def kernel(traj, mat, traj_len, emb_t, emb_l, emb_u, emb_su, emb_sl,
           emb_tu, emb_tl):
    emb_lu = jnp.concatenate([emb_l, emb_u], axis=1)
    idx_t = ((traj[:, :, 2] - 1) % HOURS + 1).reshape(NW, NCH, CHUNK)
    idx_l = traj[:, :, 1].reshape(NW, NCH, CHUNK)
    idx_u = traj[:, :, 0].reshape(NW, NCH, CHUNK)
    joint = _joint(emb_lu, emb_t, idx_t, idx_l, idx_u).reshape(B, L, D)

    m800 = mat.reshape(B, 2 * L * L)
    dsE = m800[:, 0::4]
    dtE = m800[:, 1::4]
    dsO = m800[:, 2::4]
    dtO = m800[:, 3::4]
    delta = _delta(traj_len.reshape(B, 1), dsE, dtE, dsO, dtO, emb_su,
                   emb_sl, emb_tu, emb_tl).reshape(B, L, L, D)
    return (joint, delta)
